# baseline (speedup 1.0000x reference)
import functools

import jax
import jax.numpy as jnp
from jax import lax
from jax.experimental import pallas as pl
from jax.experimental.pallas import tpu as pltpu
from jax.experimental.pallas import tpu_sc as plsc

GRID = 32
HEIGHT = 512
WIDTH = 512
L = 16
NC = 2
NS = 16
ROWS_PER_WORKER = HEIGHT // (NC * NS)
CHUNKS = WIDTH // L
CHUNKS_PER_TILE = CHUNKS // NS


def _ranks16(line_vecs, pos):
    one = jnp.full((L,), 1, jnp.int32)
    zero = jnp.full((L,), 0, jnp.int32)
    best_d = jnp.full((L,), jnp.inf, dtype=jnp.float32)
    best_l = jnp.full((L,), -jnp.inf, dtype=jnp.float32)
    for j in range(GRID):
        lj = line_vecs[j // L][j % L]
        d = jnp.abs(pos - lj)
        strict = jnp.where(d < best_d, one, zero)
        tie_better = jnp.where(d == best_d, one, zero) * jnp.where(
            best_l > lj, one, zero)
        take = (strict + tie_better) > zero
        best_d = jnp.where(take, d, best_d)
        best_l = jnp.where(take, jnp.full((L,), lj), best_l)
    rank = jnp.zeros((L,), dtype=jnp.int32)
    for j in range(GRID):
        rank = rank + jnp.where(best_l > line_vecs[j // L][j % L], one, zero)
    return rank


def _sc_body(h_hbm, v_hbm, out_hbm, lh_ref, lv_ref, nv_ref, tmp_ref,
             outbuf_ref, shared_nv):
    c = lax.axis_index("c")
    s = lax.axis_index("s")
    wid = c * NS + s
    row_base = wid * ROWS_PER_WORKER

    pltpu.sync_copy(h_hbm, lh_ref)
    pltpu.sync_copy(v_hbm, lv_ref)
    lv_vecs = [lv_ref[pl.ds(0, L)], lv_ref[pl.ds(L, L)]]
    lh_vecs = [lh_ref[pl.ds(0, L)], lh_ref[pl.ds(L, L)]]

    lane = lax.iota(jnp.int32, L)
    for t in range(CHUNKS_PER_TILE):
        col0 = (s * CHUNKS_PER_TILE + t) * L
        pos = (lane + col0).astype(jnp.float32)
        tmp_ref[pl.ds(t * L, L)] = _ranks16(lv_vecs, pos)
    pltpu.sync_copy(tmp_ref, shared_nv.at[pl.ds(s * CHUNKS_PER_TILE * L,
                                                CHUNKS_PER_TILE * L)])
    plsc.subcore_barrier()
    pltpu.sync_copy(shared_nv, nv_ref)

    pos_h = (lane + row_base).astype(jnp.float32)
    off = _ranks16(lh_vecs, pos_h) * GRID

    for ci in range(CHUNKS):
        nv_chunk = nv_ref[pl.ds(ci * L, L)]
        for r in range(ROWS_PER_WORKER):
            outbuf_ref[r, pl.ds(ci * L, L)] = nv_chunk + off[r]

    pltpu.sync_copy(outbuf_ref, out_hbm.at[pl.ds(row_base, ROWS_PER_WORKER), :])


@functools.partial(jax.jit, static_argnums=(2, 3))
def _sc_launch(horizontal_lines, vertical_lines, height, width):
    mesh = plsc.VectorSubcoreMesh(core_axis_name="c", subcore_axis_name="s")
    return pl.kernel(
        _sc_body,
        out_type=jax.ShapeDtypeStruct((height, width), jnp.int32),
        mesh=mesh,
        scratch_types=[
            pltpu.VMEM((GRID,), jnp.float32),
            pltpu.VMEM((GRID,), jnp.float32),
            pltpu.VMEM((WIDTH,), jnp.int32),
            pltpu.VMEM((CHUNKS_PER_TILE * L,), jnp.int32),
            pltpu.VMEM((ROWS_PER_WORKER, WIDTH), jnp.int32),
            pltpu.VMEM_SHARED((WIDTH,), jnp.int32),
        ],
    )(horizontal_lines, vertical_lines)


def kernel(x, horizontal_lines, vertical_lines):
    _, _, height, width = x.shape
    return _sc_launch(horizontal_lines, vertical_lines, height, width)


# speedup vs baseline: 1.0761x; 1.0761x over previous
import functools

import jax
import jax.numpy as jnp
from jax import lax
from jax.experimental import pallas as pl
from jax.experimental.pallas import tpu as pltpu
from jax.experimental.pallas import tpu_sc as plsc

GRID = 32
HEIGHT = 512
WIDTH = 512
L = 16
NC = 2
NS = 16
ROWS_PER_WORKER = HEIGHT // (NC * NS)
CHUNKS = WIDTH // L
CHUNKS_PER_TILE = CHUNKS // NS


def _ranks16(line_vecs, pos):
    one = jnp.full((L,), 1, jnp.int32)
    zero = jnp.full((L,), 0, jnp.int32)
    best_d = jnp.full((L,), jnp.inf, dtype=jnp.float32)
    best_l = jnp.full((L,), -jnp.inf, dtype=jnp.float32)
    for j in range(GRID):
        lj = line_vecs[j // L][j % L]
        d = jnp.abs(pos - lj)
        strict = jnp.where(d < best_d, one, zero)
        tie_better = jnp.where(d == best_d, one, zero) * jnp.where(
            best_l > lj, one, zero)
        take = (strict + tie_better) > zero
        best_d = jnp.where(take, d, best_d)
        best_l = jnp.where(take, jnp.full((L,), lj), best_l)
    rank = jnp.zeros((L,), dtype=jnp.int32)
    for j in range(GRID):
        rank = rank + jnp.where(best_l > line_vecs[j // L][j % L], one, zero)
    return rank


def _sc_body(lines_hbm, out_hbm, lines_ref, nv_ref, tmp_ref,
             outbuf_ref, shared_nv):
    c = lax.axis_index("c")
    s = lax.axis_index("s")
    wid = c * NS + s
    row_base = wid * ROWS_PER_WORKER

    pltpu.sync_copy(lines_hbm, lines_ref)
    lh_vecs = [lines_ref[pl.ds(0, L)], lines_ref[pl.ds(L, L)]]
    lv_vecs = [lines_ref[pl.ds(2 * L, L)], lines_ref[pl.ds(3 * L, L)]]

    lane = lax.iota(jnp.int32, L)
    for t in range(CHUNKS_PER_TILE):
        col0 = (s * CHUNKS_PER_TILE + t) * L
        pos = (lane + col0).astype(jnp.float32)
        tmp_ref[pl.ds(t * L, L)] = _ranks16(lv_vecs, pos)
    pltpu.sync_copy(tmp_ref, shared_nv.at[pl.ds(s * CHUNKS_PER_TILE * L,
                                                CHUNKS_PER_TILE * L)])
    plsc.subcore_barrier()
    pltpu.sync_copy(shared_nv, nv_ref)

    pos_h = (lane + row_base).astype(jnp.float32)
    off = _ranks16(lh_vecs, pos_h) * GRID

    def fill_chunk(ci, _):
        nv_chunk = nv_ref[pl.ds(ci * L, L)]
        for r in range(ROWS_PER_WORKER):
            outbuf_ref[r, pl.ds(ci * L, L)] = nv_chunk + off[r]
        return 0

    lax.fori_loop(0, CHUNKS, fill_chunk, 0)

    pltpu.sync_copy(outbuf_ref, out_hbm.at[pl.ds(row_base, ROWS_PER_WORKER), :])


@functools.partial(jax.jit, static_argnums=(2, 3))
def _sc_launch(horizontal_lines, vertical_lines, height, width):
    mesh = plsc.VectorSubcoreMesh(core_axis_name="c", subcore_axis_name="s")
    return pl.kernel(
        _sc_body,
        out_type=jax.ShapeDtypeStruct((height, width), jnp.int32),
        mesh=mesh,
        scratch_types=[
            pltpu.VMEM((2 * GRID,), jnp.float32),
            pltpu.VMEM((WIDTH,), jnp.int32),
            pltpu.VMEM((CHUNKS_PER_TILE * L,), jnp.int32),
            pltpu.VMEM((ROWS_PER_WORKER, WIDTH), jnp.int32),
            pltpu.VMEM_SHARED((WIDTH,), jnp.int32),
        ],
    )(jnp.concatenate([horizontal_lines, vertical_lines]))


def kernel(x, horizontal_lines, vertical_lines):
    _, _, height, width = x.shape
    return _sc_launch(horizontal_lines, vertical_lines, height, width)
